# Initial kernel scaffold; baseline (speedup 1.0000x reference)
import jax
import jax.numpy as jnp
from jax.experimental import pallas as pl


def kernel(query, pos_result, neg_result, query_len, pos_len, neg_len, emb_table):
    raise NotImplementedError("write your pallas kernel here")


# speedup vs baseline: 1.9591x; 1.9591x over previous
import functools

import jax
import jax.numpy as jnp
from jax import lax
from jax.experimental import pallas as pl
from jax.experimental.pallas import tpu as pltpu
from jax.experimental.pallas import tpu_sc as plsc

_B = 4096
_L = 50
_LP = 56
_D = 64
_NC = 2
_NS = 16
_NW = _NC * _NS
_BPW = _B // _NW


def _sc_dots(idx_all, table):
    mesh = plsc.VectorSubcoreMesh(core_axis_name="c", subcore_axis_name="s")

    @functools.partial(
        pl.kernel,
        mesh=mesh,
        out_type=(
            jax.ShapeDtypeStruct((_B,), jnp.float32),
            jax.ShapeDtypeStruct((_B,), jnp.float32),
        ),
        scratch_types=[
            pltpu.VMEM((3 * _BPW, _LP), jnp.int32),
            pltpu.VMEM((_LP, _D), jnp.float32),
            pltpu.VMEM((_BPW,), jnp.float32),
            pltpu.VMEM((_BPW,), jnp.float32),
            pltpu.VMEM((16,), jnp.float32),
            pltpu.SemaphoreType.DMA,
        ],
        compiler_params=pltpu.CompilerParams(
            needs_layout_passes=False, use_tc_tiling_on_sc=False),
    )
    def k(idx_hbm, table_hbm, s1_hbm, s2_hbm, idx_v, rows_v, s1_v, s2_v, red_v,
          sem):
        wid = lax.axis_index("s") * _NC + lax.axis_index("c")
        base = wid * (3 * _BPW)
        pltpu.sync_copy(idx_hbm.at[pl.ds(base, 3 * _BPW), :], idx_v)

        def body(bl, carry):
            sums = []
            for s in range(3):
                pltpu.async_copy(
                    table_hbm.at[idx_v.at[bl * 3 + s]], rows_v, sem
                ).wait()
                accs = [rows_v[0, pl.ds(c * 16, 16)] for c in range(4)]
                for r in range(1, _L):
                    for c in range(4):
                        accs[c] = accs[c] + rows_v[r, pl.ds(c * 16, 16)]
                sums.append(accs)
            t1 = sums[0][0] * sums[1][0]
            t2 = sums[0][0] * sums[2][0]
            for c in range(1, 4):
                t1 = t1 + sums[0][c] * sums[1][c]
                t2 = t2 + sums[0][c] * sums[2][c]
            def _xsum(v):
                for sh in (8, 4, 2, 1):
                    red_v[...] = v
                    perm = lax.iota(jnp.int32, 16) ^ sh
                    v = v + plsc.load_gather(red_v, [perm])
                return v

            lane0 = lax.iota(jnp.int32, 16) == 0
            blv = jnp.full((16,), bl, jnp.int32)
            plsc.store_scatter(s1_v, [blv], _xsum(t1), mask=lane0)
            plsc.store_scatter(s2_v, [blv], _xsum(t2), mask=lane0)
            return carry

        lax.fori_loop(0, _BPW, body, 0)
        pltpu.sync_copy(s1_v, s1_hbm.at[pl.ds(wid * _BPW, _BPW)])
        pltpu.sync_copy(s2_v, s2_hbm.at[pl.ds(wid * _BPW, _BPW)])

    return k(idx_all, table)


def _tc_loss(s1, s2, iq, ip, inn):
    def body(s1_ref, s2_ref, iq_ref, ip_ref, in_ref, out_ref):
        x = iq_ref[...] * (ip_ref[...] * s1_ref[...] - in_ref[...] * s2_ref[...])
        y = jnp.maximum(-x, 0.0) + jnp.log1p(jnp.exp(-jnp.abs(x)))
        out_ref[...] = (jnp.sum(y) / _B).reshape(1, 1)

    return pl.pallas_call(
        body,
        out_shape=jax.ShapeDtypeStruct((1, 1), jnp.float32),
    )(s1, s2, iq, ip, inn)


def kernel(query, pos_result, neg_result, query_len, pos_len, neg_len, emb_table):
    idx = jnp.stack(
        [query.astype(jnp.int32), pos_result.astype(jnp.int32),
         neg_result.astype(jnp.int32)], axis=1)
    idx = jnp.pad(idx, ((0, 0), (0, 0), (0, _LP - _L)))
    idx = idx.reshape(_B * 3, _LP)

    def _inv(l):
        return (1.0 / jnp.maximum(l, 1).astype(jnp.float32)).reshape(32, 128)

    s1, s2 = _sc_dots(idx, emb_table)
    loss = _tc_loss(s1.reshape(32, 128), s2.reshape(32, 128),
                    _inv(query_len), _inv(pos_len), _inv(neg_len))
    return loss[0, 0]
